# baseline (speedup 1.0000x reference)
import functools

import jax
import jax.numpy as jnp
from jax import lax
from jax.experimental import pallas as pl
from jax.experimental.pallas import tpu as pltpu
from jax.experimental.pallas import tpu_sc as plsc

ATOM_FDIM = 39
BOND_FDIM = 11
HIDDEN = 128
DEPTH = 6
N_ATOMS = 50000
N_BONDS = 100000
MAX_NB = 6
N_MOLS = 2000
MOL_LEN = 25

NUM_WORKERS = 32
SC_CHUNK = 256
NIDX = MAX_NB * SC_CHUNK
N_BONDS_PAD = 102400
N_ATOMS_PAD = 51200
BOND_BLK = 2048
ATOM_BLK = 6400
N_MOLS_PAD = N_ATOMS_PAD // MOL_LEN


def _gather_sum_sc(message, idx_c):
    chunk_lo, chunk_hi = 0, idx_c.shape[0]
    n_local = chunk_hi - chunk_lo
    per_worker = (n_local + NUM_WORKERS - 1) // NUM_WORKERS
    mesh = plsc.VectorSubcoreMesh(core_axis_name="c", subcore_axis_name="s")

    @functools.partial(
        pl.kernel,
        out_type=jax.ShapeDtypeStruct((n_local * SC_CHUNK, HIDDEN),
                                      jnp.float32),
        mesh=mesh,
        scratch_types=[
            pltpu.VMEM((per_worker * NIDX,), jnp.int32),
            pltpu.VMEM((SC_CHUNK, HIDDEN), jnp.float32),
            pltpu.VMEM((SC_CHUNK, HIDDEN), jnp.float32),
            pltpu.SemaphoreType.DMA,
            [pltpu.SemaphoreType.DMA] * 2,
            [pltpu.SemaphoreType.DMA] * 2,
            [pltpu.SemaphoreType.DMA] * 2,
        ],
    )
    def k(msg_hbm, idx_hbm, out_hbm, idx_all, acc_a, acc_b,
          semx, semi, sema, sems):
        wid = lax.axis_index("s") * 2 + lax.axis_index("c")
        accs = [acc_a, acc_b]

        def cid(k_):
            return chunk_lo + wid + k_ * NUM_WORKERS

        def valid(k_):
            return cid(k_) < chunk_hi

        def obase(k_):
            return (cid(k_) - chunk_lo) * SC_CHUNK

        def idx_slice(k_, j):
            return idx_all.at[pl.ds(k_ * NIDX + j * SC_CHUNK, SC_CHUNK)]

        def init_gather(k_):
            pltpu.async_copy(
                msg_hbm.at[idx_slice(k_, 0)], accs[k_ % 2], semi[k_ % 2])

        def wait_init(k_):
            pltpu.make_async_copy(
                msg_hbm.at[idx_slice(k_, 0)], accs[k_ % 2],
                semi[k_ % 2]).wait()

        def store(k_):
            pltpu.async_copy(
                accs[k_ % 2], out_hbm.at[pl.ds(obase(k_), SC_CHUNK)],
                sems[k_ % 2])

        def wait_store(k_):
            pltpu.make_async_copy(
                accs[k_ % 2], out_hbm.at[pl.ds(obase(k_), SC_CHUNK)],
                sems[k_ % 2]).wait()

        for k_ in range(per_worker):
            @pl.when(valid(k_))
            def _(k_=k_):
                pltpu.async_copy(
                    idx_hbm.at[cid(k_)],
                    idx_all.at[pl.ds(k_ * NIDX, NIDX)], semx)
        for k_ in range(per_worker):
            @pl.when(valid(k_))
            def _(k_=k_):
                pltpu.make_async_copy(
                    idx_hbm.at[cid(k_)],
                    idx_all.at[pl.ds(k_ * NIDX, NIDX)], semx).wait()

        @pl.when(valid(0))
        def _():
            init_gather(0)

        for k_ in range(per_worker):
            p = k_ % 2

            @pl.when(valid(k_))
            def _(k_=k_, p=p):
                wait_init(k_)
                for j in range(1, MAX_NB):
                    pltpu.async_copy(
                        msg_hbm.at[idx_slice(k_, j)], accs[p], sema[p],
                        add=True)

            if k_ + 1 < per_worker:
                @pl.when(valid(k_ + 1))
                def _(k_=k_):
                    if k_ >= 1:
                        wait_store(k_ - 1)
                    init_gather(k_ + 1)

            @pl.when(valid(k_))
            def _(k_=k_, p=p):
                for j in range(1, MAX_NB):
                    pltpu.make_async_copy(
                        msg_hbm.at[idx_slice(k_, j)], accs[p],
                        sema[p]).wait()
                store(k_)

        for k_ in range(per_worker):
            @pl.when(valid(k_) & (cid(k_) + 2 * NUM_WORKERS >= chunk_hi))
            def _(k_=k_):
                wait_store(k_)

    return k(message, idx_c)


def _mm_init(fbonds_p, W_i):
    grid = N_BONDS_PAD // BOND_BLK

    def body(f_ref, w_ref, bin_ref, msg_ref):
        acc = lax.dot_general(f_ref[...], w_ref[...],
                              (((1,), (1,)), ((), ())),
                              preferred_element_type=jnp.float32)
        bin_ref[...] = acc
        msg_ref[...] = jnp.maximum(acc, 0.0)

    return pl.pallas_call(
        body,
        grid=(grid,),
        in_specs=[
            pl.BlockSpec((BOND_BLK, ATOM_FDIM + BOND_FDIM),
                         lambda i: (i, 0)),
            pl.BlockSpec((HIDDEN, ATOM_FDIM + BOND_FDIM), lambda i: (0, 0)),
        ],
        out_specs=[
            pl.BlockSpec((BOND_BLK, HIDDEN), lambda i: (i, 0)),
            pl.BlockSpec((BOND_BLK, HIDDEN), lambda i: (i, 0)),
        ],
        out_shape=[
            jax.ShapeDtypeStruct((N_BONDS_PAD, HIDDEN), jnp.float32),
            jax.ShapeDtypeStruct((N_BONDS_PAD, HIDDEN), jnp.float32),
        ],
    )(fbonds_p, W_i)


def _mm_h_part(nei_part, binput, W_h, blk_off, prev=None):
    grid = nei_part.shape[0] // BOND_BLK

    def body(*refs):
        n_ref, b_ref, w_ref = refs[0], refs[1], refs[2]
        o_ref = refs[-1]
        acc = lax.dot_general(n_ref[...], w_ref[...],
                              (((1,), (1,)), ((), ())),
                              preferred_element_type=jnp.float32)
        o_ref[...] = jnp.maximum(b_ref[...] + acc, 0.0)

    in_specs = [
        pl.BlockSpec((BOND_BLK, HIDDEN), lambda i: (i, 0)),
        pl.BlockSpec((BOND_BLK, HIDDEN), lambda i: (i + blk_off, 0)),
        pl.BlockSpec((HIDDEN, HIDDEN), lambda i: (0, 0)),
    ]
    args = [nei_part, binput, W_h]
    aliases = {}
    if prev is not None:
        in_specs.append(pl.BlockSpec((8, HIDDEN), lambda i: (0, 0)))
        args.append(prev)
        aliases = {3: 0}

    return pl.pallas_call(
        body,
        grid=(grid,),
        in_specs=in_specs,
        out_specs=pl.BlockSpec((BOND_BLK, HIDDEN),
                               lambda i: (i + blk_off, 0)),
        out_shape=jax.ShapeDtypeStruct((N_BONDS_PAD, HIDDEN), jnp.float32),
        input_output_aliases=aliases,
    )(*args)


def _final_part(fatoms_p, nei_part, W_oa, W_om, W_o_b, blk_off, prev=None):
    grid = nei_part.shape[0] // ATOM_BLK
    mb = ATOM_BLK // MOL_LEN

    def body(*refs):
        f_ref, n_ref, wa_ref, wm_ref, b_ref = refs[:5]
        o_ref = refs[-1]
        h = lax.dot_general(f_ref[...], wa_ref[...],
                            (((1,), (1,)), ((), ())),
                            preferred_element_type=jnp.float32)
        h = h + lax.dot_general(n_ref[...], wm_ref[...],
                                (((1,), (1,)), ((), ())),
                                preferred_element_type=jnp.float32)
        h = jnp.maximum(h + b_ref[...], 0.0)
        mrow = lax.broadcasted_iota(jnp.int32, (mb, ATOM_BLK), 0)
        acol = lax.broadcasted_iota(jnp.int32, (mb, ATOM_BLK), 1) // MOL_LEN
        P = jnp.where(mrow == acol, 1.0 / MOL_LEN, 0.0).astype(jnp.float32)
        o_ref[...] = lax.dot_general(P, h, (((1,), (0,)), ((), ())),
                                     preferred_element_type=jnp.float32)

    in_specs = [
        pl.BlockSpec((ATOM_BLK, ATOM_FDIM), lambda i: (i + blk_off, 0)),
        pl.BlockSpec((ATOM_BLK, HIDDEN), lambda i: (i, 0)),
        pl.BlockSpec((HIDDEN, ATOM_FDIM), lambda i: (0, 0)),
        pl.BlockSpec((HIDDEN, HIDDEN), lambda i: (0, 0)),
        pl.BlockSpec((1, HIDDEN), lambda i: (0, 0)),
    ]
    args = [fatoms_p, nei_part, W_oa, W_om, W_o_b]
    aliases = {}
    if prev is not None:
        in_specs.append(pl.BlockSpec((8, HIDDEN), lambda i: (0, 0)))
        args.append(prev)
        aliases = {5: 0}

    return pl.pallas_call(
        body,
        grid=(grid,),
        in_specs=in_specs,
        out_specs=pl.BlockSpec((mb, HIDDEN), lambda i: (i + blk_off, 0)),
        out_shape=jax.ShapeDtypeStruct((N_MOLS_PAD, HIDDEN), jnp.float32),
        input_output_aliases=aliases,
    )(*args)


def _chunk_indices(graph, n_rows_pad):
    g = jnp.pad(graph, ((0, n_rows_pad - graph.shape[0]), (0, 0)))
    g = g.reshape(n_rows_pad // SC_CHUNK, SC_CHUNK, MAX_NB)
    return g.transpose(0, 2, 1).reshape(-1, NIDX)


def kernel(fatoms, fbonds, agraph, bgraph, scope, W_i, W_h, W_o_w, W_o_b):
    del scope
    bidx = _chunk_indices(bgraph, N_BONDS_PAD)
    aidx = _chunk_indices(agraph, N_ATOMS_PAD)
    fbonds_p = jnp.pad(fbonds, ((0, N_BONDS_PAD - N_BONDS), (0, 0)))
    fatoms_p = jnp.pad(fatoms, ((0, N_ATOMS_PAD - N_ATOMS), (0, 0)))

    n_bchunks = N_BONDS_PAD // SC_CHUNK
    n_achunks = N_ATOMS_PAD // SC_CHUNK
    half_b = n_bchunks // 2
    half_a = n_achunks // 2
    half_b_blk = (half_b * SC_CHUNK) // BOND_BLK
    half_a_blk = (half_a * SC_CHUNK) // ATOM_BLK

    bidx0, bidx1 = bidx[:half_b], bidx[half_b:]
    aidx0, aidx1 = aidx[:half_a], aidx[half_a:]
    binput, message = _mm_init(fbonds_p, W_i)
    for _ in range(DEPTH - 1):
        n1 = _gather_sum_sc(message, bidx1)
        n0 = _gather_sum_sc(message, bidx0)
        m1 = _mm_h_part(n1, binput, W_h, half_b_blk)
        message = _mm_h_part(n0, binput, W_h, 0, prev=m1)
    a0 = _gather_sum_sc(message, aidx0)
    a1 = _gather_sum_sc(message, aidx1)
    W_oa = W_o_w[:, :ATOM_FDIM]
    W_om = W_o_w[:, ATOM_FDIM:]
    bias = W_o_b.reshape(1, HIDDEN)
    f0 = _final_part(fatoms_p, a0, W_oa, W_om, bias, 0)
    f1 = _final_part(fatoms_p, a1, W_oa, W_om, bias, half_a_blk, prev=f0)
    return f1[:N_MOLS]


# speedup vs baseline: 3.2849x; 3.2849x over previous
import functools

import jax
import jax.numpy as jnp
from jax import lax
from jax.experimental import pallas as pl
from jax.experimental.pallas import tpu as pltpu
from jax.experimental.pallas import tpu_sc as plsc

ATOM_FDIM = 39
BOND_FDIM = 11
HIDDEN = 128
DEPTH = 6
N_ATOMS = 50000
N_BONDS = 100000
MAX_NB = 6
N_MOLS = 2000
MOL_LEN = 25

NUM_WORKERS = 32
SC_CHUNK = 256
NIDX = MAX_NB * SC_CHUNK
N_BONDS_PAD = 102400
N_ATOMS_PAD = 51200
BOND_BLK = 2048
ATOM_BLK = 6400
N_MOLS_PAD = N_ATOMS_PAD // MOL_LEN


def _gather_sum_sc(message, idx_c):
    chunk_lo, chunk_hi = 0, idx_c.shape[0]
    n_local = chunk_hi - chunk_lo
    per_worker = (n_local + NUM_WORKERS - 1) // NUM_WORKERS
    mesh = plsc.VectorSubcoreMesh(core_axis_name="c", subcore_axis_name="s")

    @functools.partial(
        pl.kernel,
        out_type=jax.ShapeDtypeStruct((n_local * SC_CHUNK, HIDDEN),
                                      jnp.float32),
        mesh=mesh,
        scratch_types=[
            pltpu.VMEM((per_worker * NIDX,), jnp.int32),
            pltpu.VMEM((SC_CHUNK, HIDDEN), jnp.float32),
            pltpu.VMEM((SC_CHUNK, HIDDEN), jnp.float32),
            pltpu.SemaphoreType.DMA,
            [pltpu.SemaphoreType.DMA] * 2,
            [pltpu.SemaphoreType.DMA] * 2,
            [pltpu.SemaphoreType.DMA] * 2,
        ],
    )
    def k(msg_hbm, idx_hbm, out_hbm, idx_all, acc_a, acc_b,
          semx, semi, sema, sems):
        wid = lax.axis_index("s") * 2 + lax.axis_index("c")
        accs = [acc_a, acc_b]

        def cid(k_):
            return chunk_lo + wid + k_ * NUM_WORKERS

        def valid(k_):
            return cid(k_) < chunk_hi

        def obase(k_):
            return (cid(k_) - chunk_lo) * SC_CHUNK

        def idx_slice(k_, j):
            return idx_all.at[pl.ds(k_ * NIDX + j * SC_CHUNK, SC_CHUNK)]

        def init_gather(k_):
            pltpu.async_copy(
                msg_hbm.at[idx_slice(k_, 0)], accs[k_ % 2], semi[k_ % 2])

        def wait_init(k_):
            pltpu.make_async_copy(
                msg_hbm.at[idx_slice(k_, 0)], accs[k_ % 2],
                semi[k_ % 2]).wait()

        def store(k_):
            pltpu.async_copy(
                accs[k_ % 2], out_hbm.at[pl.ds(obase(k_), SC_CHUNK)],
                sems[k_ % 2])

        def wait_store(k_):
            pltpu.make_async_copy(
                accs[k_ % 2], out_hbm.at[pl.ds(obase(k_), SC_CHUNK)],
                sems[k_ % 2]).wait()

        for k_ in range(per_worker):
            @pl.when(valid(k_))
            def _(k_=k_):
                pltpu.async_copy(
                    idx_hbm.at[cid(k_)],
                    idx_all.at[pl.ds(k_ * NIDX, NIDX)], semx)
        for k_ in range(per_worker):
            @pl.when(valid(k_))
            def _(k_=k_):
                pltpu.make_async_copy(
                    idx_hbm.at[cid(k_)],
                    idx_all.at[pl.ds(k_ * NIDX, NIDX)], semx).wait()

        @pl.when(valid(0))
        def _():
            init_gather(0)

        for k_ in range(per_worker):
            p = k_ % 2

            @pl.when(valid(k_))
            def _(k_=k_, p=p):
                wait_init(k_)
                for j in range(1, MAX_NB):
                    pltpu.async_copy(
                        msg_hbm.at[idx_slice(k_, j)], accs[p], sema[p],
                        add=True)

            if k_ + 1 < per_worker:
                @pl.when(valid(k_ + 1))
                def _(k_=k_):
                    if k_ >= 1:
                        wait_store(k_ - 1)
                    init_gather(k_ + 1)

            @pl.when(valid(k_))
            def _(k_=k_, p=p):
                for j in range(1, MAX_NB):
                    pltpu.make_async_copy(
                        msg_hbm.at[idx_slice(k_, j)], accs[p],
                        sema[p]).wait()
                store(k_)

        for k_ in range(per_worker):
            @pl.when(valid(k_) & (cid(k_) + 2 * NUM_WORKERS >= chunk_hi))
            def _(k_=k_):
                wait_store(k_)

    return k(message, idx_c)


def _mm_init(fbonds_p, W_i):
    grid = N_BONDS_PAD // BOND_BLK

    def body(f_ref, w_ref, bin_ref, msg_ref):
        acc = lax.dot_general(f_ref[...], w_ref[...],
                              (((1,), (1,)), ((), ())),
                              preferred_element_type=jnp.float32)
        bin_ref[...] = acc
        msg_ref[...] = jnp.maximum(acc, 0.0)

    return pl.pallas_call(
        body,
        grid=(grid,),
        in_specs=[
            pl.BlockSpec((BOND_BLK, ATOM_FDIM + BOND_FDIM),
                         lambda i: (i, 0)),
            pl.BlockSpec((HIDDEN, ATOM_FDIM + BOND_FDIM), lambda i: (0, 0)),
        ],
        out_specs=[
            pl.BlockSpec((BOND_BLK, HIDDEN), lambda i: (i, 0)),
            pl.BlockSpec((BOND_BLK, HIDDEN), lambda i: (i, 0)),
        ],
        out_shape=[
            jax.ShapeDtypeStruct((N_BONDS_PAD, HIDDEN), jnp.float32),
            jax.ShapeDtypeStruct((N_BONDS_PAD, HIDDEN), jnp.float32),
        ],
    )(fbonds_p, W_i)


def _mm_h_part(nei_part, binput, W_h, blk_off, prev=None):
    grid = nei_part.shape[0] // BOND_BLK

    def body(*refs):
        n_ref, b_ref, w_ref = refs[0], refs[1], refs[2]
        o_ref = refs[-1]
        acc = lax.dot_general(n_ref[...], w_ref[...],
                              (((1,), (1,)), ((), ())),
                              preferred_element_type=jnp.float32)
        o_ref[...] = jnp.maximum(b_ref[...] + acc, 0.0)

    in_specs = [
        pl.BlockSpec((BOND_BLK, HIDDEN), lambda i: (i, 0)),
        pl.BlockSpec((BOND_BLK, HIDDEN), lambda i: (i + blk_off, 0)),
        pl.BlockSpec((HIDDEN, HIDDEN), lambda i: (0, 0)),
    ]
    args = [nei_part, binput, W_h]
    aliases = {}
    if prev is not None:
        in_specs.append(pl.BlockSpec((8, HIDDEN), lambda i: (0, 0)))
        args.append(prev)
        aliases = {3: 0}

    return pl.pallas_call(
        body,
        grid=(grid,),
        in_specs=in_specs,
        out_specs=pl.BlockSpec((BOND_BLK, HIDDEN),
                               lambda i: (i + blk_off, 0)),
        out_shape=jax.ShapeDtypeStruct((N_BONDS_PAD, HIDDEN), jnp.float32),
        input_output_aliases=aliases,
    )(*args)


def _final_part(fatoms_p, nei_part, W_oa, W_om, W_o_b, blk_off, prev=None):
    grid = nei_part.shape[0] // ATOM_BLK
    mb = ATOM_BLK // MOL_LEN

    def body(*refs):
        f_ref, n_ref, wa_ref, wm_ref, b_ref = refs[:5]
        o_ref = refs[-1]
        h = lax.dot_general(f_ref[...], wa_ref[...],
                            (((1,), (1,)), ((), ())),
                            preferred_element_type=jnp.float32)
        h = h + lax.dot_general(n_ref[...], wm_ref[...],
                                (((1,), (1,)), ((), ())),
                                preferred_element_type=jnp.float32)
        h = jnp.maximum(h + b_ref[...], 0.0)
        mrow = lax.broadcasted_iota(jnp.int32, (mb, ATOM_BLK), 0)
        acol = lax.broadcasted_iota(jnp.int32, (mb, ATOM_BLK), 1) // MOL_LEN
        P = jnp.where(mrow == acol, 1.0 / MOL_LEN, 0.0).astype(jnp.float32)
        o_ref[...] = lax.dot_general(P, h, (((1,), (0,)), ((), ())),
                                     preferred_element_type=jnp.float32)

    in_specs = [
        pl.BlockSpec((ATOM_BLK, ATOM_FDIM), lambda i: (i + blk_off, 0)),
        pl.BlockSpec((ATOM_BLK, HIDDEN), lambda i: (i, 0)),
        pl.BlockSpec((HIDDEN, ATOM_FDIM), lambda i: (0, 0)),
        pl.BlockSpec((HIDDEN, HIDDEN), lambda i: (0, 0)),
        pl.BlockSpec((1, HIDDEN), lambda i: (0, 0)),
    ]
    args = [fatoms_p, nei_part, W_oa, W_om, W_o_b]
    aliases = {}
    if prev is not None:
        in_specs.append(pl.BlockSpec((8, HIDDEN), lambda i: (0, 0)))
        args.append(prev)
        aliases = {5: 0}

    return pl.pallas_call(
        body,
        grid=(grid,),
        in_specs=in_specs,
        out_specs=pl.BlockSpec((mb, HIDDEN), lambda i: (i + blk_off, 0)),
        out_shape=jax.ShapeDtypeStruct((N_MOLS_PAD, HIDDEN), jnp.float32),
        input_output_aliases=aliases,
    )(*args)


def _chunk_indices(graph, n_rows_pad, n_table):
    pad = n_rows_pad - graph.shape[0]
    filler = jnp.arange(pad * MAX_NB, dtype=jnp.int32) % n_table
    g = jnp.concatenate([graph, filler.reshape(pad, MAX_NB)], axis=0)
    g = g.reshape(n_rows_pad // SC_CHUNK, SC_CHUNK, MAX_NB)
    return g.transpose(0, 2, 1).reshape(-1, NIDX)


def kernel(fatoms, fbonds, agraph, bgraph, scope, W_i, W_h, W_o_w, W_o_b):
    del scope
    bidx = _chunk_indices(bgraph, N_BONDS_PAD, N_BONDS)
    aidx = _chunk_indices(agraph, N_ATOMS_PAD, N_BONDS)
    fbonds_p = jnp.pad(fbonds, ((0, N_BONDS_PAD - N_BONDS), (0, 0)))
    fatoms_p = jnp.pad(fatoms, ((0, N_ATOMS_PAD - N_ATOMS), (0, 0)))

    n_bchunks = N_BONDS_PAD // SC_CHUNK
    n_achunks = N_ATOMS_PAD // SC_CHUNK
    half_b = n_bchunks // 2
    half_a = n_achunks // 2
    half_b_blk = (half_b * SC_CHUNK) // BOND_BLK
    half_a_blk = (half_a * SC_CHUNK) // ATOM_BLK

    bidx0, bidx1 = bidx[:half_b], bidx[half_b:]
    aidx0, aidx1 = aidx[:half_a], aidx[half_a:]
    binput, message = _mm_init(fbonds_p, W_i)
    for _ in range(DEPTH - 1):
        n0 = _gather_sum_sc(message, bidx0)
        n1 = _gather_sum_sc(message, bidx1)
        m0 = _mm_h_part(n0, binput, W_h, 0)
        message = _mm_h_part(n1, binput, W_h, half_b_blk, prev=m0)
    a0 = _gather_sum_sc(message, aidx0)
    a1 = _gather_sum_sc(message, aidx1)
    W_oa = W_o_w[:, :ATOM_FDIM]
    W_om = W_o_w[:, ATOM_FDIM:]
    bias = W_o_b.reshape(1, HIDDEN)
    f0 = _final_part(fatoms_p, a0, W_oa, W_om, bias, 0)
    f1 = _final_part(fatoms_p, a1, W_oa, W_om, bias, half_a_blk, prev=f0)
    return f1[:N_MOLS]


# speedup vs baseline: 3.3569x; 1.0219x over previous
import functools

import jax
import jax.numpy as jnp
from jax import lax
from jax.experimental import pallas as pl
from jax.experimental.pallas import tpu as pltpu
from jax.experimental.pallas import tpu_sc as plsc

ATOM_FDIM = 39
BOND_FDIM = 11
HIDDEN = 128
DEPTH = 6
N_ATOMS = 50000
N_BONDS = 100000
MAX_NB = 6
N_MOLS = 2000
MOL_LEN = 25

NUM_WORKERS = 32
SC_CHUNK = 256
NIDX = MAX_NB * SC_CHUNK
N_BONDS_PAD = 102400
N_ATOMS_PAD = 51200
BOND_BLK = 2048
ATOM_BLK = 6400
N_MOLS_PAD = N_ATOMS_PAD // MOL_LEN


def _gather_sum_sc(message, idx_c):
    chunk_lo, chunk_hi = 0, idx_c.shape[0]
    n_local = chunk_hi - chunk_lo
    per_worker = (n_local + NUM_WORKERS - 1) // NUM_WORKERS
    mesh = plsc.VectorSubcoreMesh(core_axis_name="c", subcore_axis_name="s")

    @functools.partial(
        pl.kernel,
        out_type=jax.ShapeDtypeStruct((n_local * SC_CHUNK, HIDDEN),
                                      jnp.float32),
        mesh=mesh,
        scratch_types=[
            pltpu.VMEM((per_worker * NIDX,), jnp.int32),
            pltpu.VMEM((SC_CHUNK, HIDDEN), jnp.float32),
            pltpu.VMEM((SC_CHUNK, HIDDEN), jnp.float32),
            pltpu.SemaphoreType.DMA,
            [pltpu.SemaphoreType.DMA] * 2,
            [pltpu.SemaphoreType.DMA] * 2,
            [pltpu.SemaphoreType.DMA] * 2,
        ],
    )
    def k(msg_hbm, idx_hbm, out_hbm, idx_all, acc_a, acc_b,
          semx, semi, sema, sems):
        wid = lax.axis_index("s") * 2 + lax.axis_index("c")
        accs = [acc_a, acc_b]

        def cid(k_):
            return chunk_lo + wid + k_ * NUM_WORKERS

        def valid(k_):
            return cid(k_) < chunk_hi

        def obase(k_):
            return (cid(k_) - chunk_lo) * SC_CHUNK

        def idx_slice(k_, j):
            return idx_all.at[pl.ds(k_ * NIDX + j * SC_CHUNK, SC_CHUNK)]

        def init_gather(k_):
            pltpu.async_copy(
                msg_hbm.at[idx_slice(k_, 0)], accs[k_ % 2], semi[k_ % 2])

        def wait_init(k_):
            pltpu.make_async_copy(
                msg_hbm.at[idx_slice(k_, 0)], accs[k_ % 2],
                semi[k_ % 2]).wait()

        def store(k_):
            pltpu.async_copy(
                accs[k_ % 2], out_hbm.at[pl.ds(obase(k_), SC_CHUNK)],
                sems[k_ % 2])

        def wait_store(k_):
            pltpu.make_async_copy(
                accs[k_ % 2], out_hbm.at[pl.ds(obase(k_), SC_CHUNK)],
                sems[k_ % 2]).wait()

        for k_ in range(per_worker):
            @pl.when(valid(k_))
            def _(k_=k_):
                pltpu.async_copy(
                    idx_hbm.at[cid(k_)],
                    idx_all.at[pl.ds(k_ * NIDX, NIDX)], semx)
        for k_ in range(per_worker):
            @pl.when(valid(k_))
            def _(k_=k_):
                pltpu.make_async_copy(
                    idx_hbm.at[cid(k_)],
                    idx_all.at[pl.ds(k_ * NIDX, NIDX)], semx).wait()

        @pl.when(valid(0))
        def _():
            init_gather(0)

        for k_ in range(per_worker):
            p = k_ % 2

            @pl.when(valid(k_))
            def _(k_=k_, p=p):
                wait_init(k_)
                for j in range(1, MAX_NB):
                    pltpu.async_copy(
                        msg_hbm.at[idx_slice(k_, j)], accs[p], sema[p],
                        add=True)

            if k_ + 1 < per_worker:
                @pl.when(valid(k_ + 1))
                def _(k_=k_):
                    if k_ >= 1:
                        wait_store(k_ - 1)
                    init_gather(k_ + 1)

            @pl.when(valid(k_))
            def _(k_=k_, p=p):
                for j in range(1, MAX_NB):
                    pltpu.make_async_copy(
                        msg_hbm.at[idx_slice(k_, j)], accs[p],
                        sema[p]).wait()
                store(k_)

        for k_ in range(per_worker):
            @pl.when(valid(k_) & (cid(k_) + 2 * NUM_WORKERS >= chunk_hi))
            def _(k_=k_):
                wait_store(k_)

    return k(message, idx_c)


def _mm_init(fbonds_p, W_i):
    grid = N_BONDS_PAD // BOND_BLK

    def body(f_ref, w_ref, bin_ref, msg_ref):
        acc = lax.dot_general(f_ref[...], w_ref[...],
                              (((1,), (1,)), ((), ())),
                              preferred_element_type=jnp.float32)
        bin_ref[...] = acc.astype(jnp.bfloat16)
        msg_ref[...] = jnp.maximum(acc, 0.0)

    return pl.pallas_call(
        body,
        grid=(grid,),
        in_specs=[
            pl.BlockSpec((BOND_BLK, ATOM_FDIM + BOND_FDIM),
                         lambda i: (i, 0)),
            pl.BlockSpec((HIDDEN, ATOM_FDIM + BOND_FDIM), lambda i: (0, 0)),
        ],
        out_specs=[
            pl.BlockSpec((BOND_BLK, HIDDEN), lambda i: (i, 0)),
            pl.BlockSpec((BOND_BLK, HIDDEN), lambda i: (i, 0)),
        ],
        out_shape=[
            jax.ShapeDtypeStruct((N_BONDS_PAD, HIDDEN), jnp.bfloat16),
            jax.ShapeDtypeStruct((N_BONDS_PAD, HIDDEN), jnp.float32),
        ],
    )(fbonds_p, W_i)


def _mm_h_part(nei_part, binput, W_h, blk_off, prev=None):
    grid = nei_part.shape[0] // BOND_BLK

    def body(*refs):
        n_ref, b_ref, w_ref = refs[0], refs[1], refs[2]
        o_ref = refs[-1]
        acc = lax.dot_general(n_ref[...], w_ref[...],
                              (((1,), (1,)), ((), ())),
                              preferred_element_type=jnp.float32)
        o_ref[...] = jnp.maximum(b_ref[...].astype(jnp.float32) + acc, 0.0)

    in_specs = [
        pl.BlockSpec((BOND_BLK, HIDDEN), lambda i: (i, 0)),
        pl.BlockSpec((BOND_BLK, HIDDEN), lambda i: (i + blk_off, 0)),
        pl.BlockSpec((HIDDEN, HIDDEN), lambda i: (0, 0)),
    ]
    args = [nei_part, binput, W_h]
    aliases = {}
    if prev is not None:
        in_specs.append(pl.BlockSpec((8, HIDDEN), lambda i: (0, 0)))
        args.append(prev)
        aliases = {3: 0}

    return pl.pallas_call(
        body,
        grid=(grid,),
        in_specs=in_specs,
        out_specs=pl.BlockSpec((BOND_BLK, HIDDEN),
                               lambda i: (i + blk_off, 0)),
        out_shape=jax.ShapeDtypeStruct((N_BONDS_PAD, HIDDEN), jnp.float32),
        input_output_aliases=aliases,
    )(*args)


def _final_part(fatoms_p, nei_part, W_oa, W_om, W_o_b, blk_off, prev=None):
    grid = nei_part.shape[0] // ATOM_BLK
    mb = ATOM_BLK // MOL_LEN

    def body(*refs):
        f_ref, n_ref, wa_ref, wm_ref, b_ref = refs[:5]
        o_ref = refs[-1]
        h = lax.dot_general(f_ref[...], wa_ref[...],
                            (((1,), (1,)), ((), ())),
                            preferred_element_type=jnp.float32)
        h = h + lax.dot_general(n_ref[...], wm_ref[...],
                                (((1,), (1,)), ((), ())),
                                preferred_element_type=jnp.float32)
        h = jnp.maximum(h + b_ref[...], 0.0)
        mrow = lax.broadcasted_iota(jnp.int32, (mb, ATOM_BLK), 0)
        acol = lax.broadcasted_iota(jnp.int32, (mb, ATOM_BLK), 1) // MOL_LEN
        P = jnp.where(mrow == acol, 1.0 / MOL_LEN, 0.0).astype(jnp.float32)
        o_ref[...] = lax.dot_general(P, h, (((1,), (0,)), ((), ())),
                                     preferred_element_type=jnp.float32)

    in_specs = [
        pl.BlockSpec((ATOM_BLK, ATOM_FDIM), lambda i: (i + blk_off, 0)),
        pl.BlockSpec((ATOM_BLK, HIDDEN), lambda i: (i, 0)),
        pl.BlockSpec((HIDDEN, ATOM_FDIM), lambda i: (0, 0)),
        pl.BlockSpec((HIDDEN, HIDDEN), lambda i: (0, 0)),
        pl.BlockSpec((1, HIDDEN), lambda i: (0, 0)),
    ]
    args = [fatoms_p, nei_part, W_oa, W_om, W_o_b]
    aliases = {}
    if prev is not None:
        in_specs.append(pl.BlockSpec((8, HIDDEN), lambda i: (0, 0)))
        args.append(prev)
        aliases = {5: 0}

    return pl.pallas_call(
        body,
        grid=(grid,),
        in_specs=in_specs,
        out_specs=pl.BlockSpec((mb, HIDDEN), lambda i: (i + blk_off, 0)),
        out_shape=jax.ShapeDtypeStruct((N_MOLS_PAD, HIDDEN), jnp.float32),
        input_output_aliases=aliases,
    )(*args)


def _chunk_indices(graph, n_rows_pad, n_table):
    pad = n_rows_pad - graph.shape[0]
    filler = jnp.arange(pad * MAX_NB, dtype=jnp.int32) % n_table
    g = jnp.concatenate([graph, filler.reshape(pad, MAX_NB)], axis=0)
    g = g.reshape(n_rows_pad // SC_CHUNK, SC_CHUNK, MAX_NB)
    return g.transpose(0, 2, 1).reshape(-1, NIDX)


def kernel(fatoms, fbonds, agraph, bgraph, scope, W_i, W_h, W_o_w, W_o_b):
    del scope
    bidx = _chunk_indices(bgraph, N_BONDS_PAD, N_BONDS)
    aidx = _chunk_indices(agraph, N_ATOMS_PAD, N_BONDS)
    fbonds_p = jnp.pad(fbonds, ((0, N_BONDS_PAD - N_BONDS), (0, 0)))
    fatoms_p = jnp.pad(fatoms, ((0, N_ATOMS_PAD - N_ATOMS), (0, 0)))

    n_bchunks = N_BONDS_PAD // SC_CHUNK
    n_achunks = N_ATOMS_PAD // SC_CHUNK
    half_b = n_bchunks // 2
    half_a = n_achunks // 2
    half_b_blk = (half_b * SC_CHUNK) // BOND_BLK
    half_a_blk = (half_a * SC_CHUNK) // ATOM_BLK

    bidx0, bidx1 = bidx[:half_b], bidx[half_b:]
    aidx0, aidx1 = aidx[:half_a], aidx[half_a:]
    binput, message = _mm_init(fbonds_p, W_i)
    for _ in range(DEPTH - 1):
        n0 = _gather_sum_sc(message, bidx0)
        n1 = _gather_sum_sc(message, bidx1)
        m0 = _mm_h_part(n0, binput, W_h, 0)
        message = _mm_h_part(n1, binput, W_h, half_b_blk, prev=m0)
    a0 = _gather_sum_sc(message, aidx0)
    a1 = _gather_sum_sc(message, aidx1)
    W_oa = W_o_w[:, :ATOM_FDIM]
    W_om = W_o_w[:, ATOM_FDIM:]
    bias = W_o_b.reshape(1, HIDDEN)
    f0 = _final_part(fatoms_p, a0, W_oa, W_om, bias, 0)
    f1 = _final_part(fatoms_p, a1, W_oa, W_om, bias, half_a_blk, prev=f0)
    return f1[:N_MOLS]


# speedup vs baseline: 3.5147x; 1.0470x over previous
import functools

import jax
import jax.numpy as jnp
from jax import lax
from jax.experimental import pallas as pl
from jax.experimental.pallas import tpu as pltpu
from jax.experimental.pallas import tpu_sc as plsc

ATOM_FDIM = 39
BOND_FDIM = 11
HIDDEN = 128
DEPTH = 6
N_ATOMS = 50000
N_BONDS = 100000
MAX_NB = 6
N_MOLS = 2000
MOL_LEN = 25

NUM_WORKERS = 32
SC_CHUNK = 256
NIDX = MAX_NB * SC_CHUNK
N_BONDS_PAD = 102400
N_ATOMS_PAD = 51200
BOND_BLK = 2048
ATOM_BLK = 3200
N_MOLS_PAD = N_ATOMS_PAD // MOL_LEN
B_SPLIT = 280
A_SPLIT = 150


def _gather_sum_sc(message, idx_c):
    chunk_lo, chunk_hi = 0, idx_c.shape[0]
    n_local = chunk_hi - chunk_lo
    per_worker = (n_local + NUM_WORKERS - 1) // NUM_WORKERS
    mesh = plsc.VectorSubcoreMesh(core_axis_name="c", subcore_axis_name="s")

    @functools.partial(
        pl.kernel,
        out_type=jax.ShapeDtypeStruct((n_local * SC_CHUNK, HIDDEN),
                                      jnp.float32),
        mesh=mesh,
        scratch_types=[
            pltpu.VMEM((per_worker * NIDX,), jnp.int32),
            pltpu.VMEM((SC_CHUNK, HIDDEN), jnp.float32),
            pltpu.VMEM((SC_CHUNK, HIDDEN), jnp.float32),
            pltpu.SemaphoreType.DMA,
            [pltpu.SemaphoreType.DMA] * 2,
            [pltpu.SemaphoreType.DMA] * 2,
            [pltpu.SemaphoreType.DMA] * 2,
        ],
    )
    def k(msg_hbm, idx_hbm, out_hbm, idx_all, acc_a, acc_b,
          semx, semi, sema, sems):
        wid = lax.axis_index("s") * 2 + lax.axis_index("c")
        accs = [acc_a, acc_b]

        def cid(k_):
            return chunk_lo + wid + k_ * NUM_WORKERS

        def valid(k_):
            return cid(k_) < chunk_hi

        def obase(k_):
            return (cid(k_) - chunk_lo) * SC_CHUNK

        def idx_slice(k_, j):
            return idx_all.at[pl.ds(k_ * NIDX + j * SC_CHUNK, SC_CHUNK)]

        def init_gather(k_):
            pltpu.async_copy(
                msg_hbm.at[idx_slice(k_, 0)], accs[k_ % 2], semi[k_ % 2])

        def wait_init(k_):
            pltpu.make_async_copy(
                msg_hbm.at[idx_slice(k_, 0)], accs[k_ % 2],
                semi[k_ % 2]).wait()

        def store(k_):
            pltpu.async_copy(
                accs[k_ % 2], out_hbm.at[pl.ds(obase(k_), SC_CHUNK)],
                sems[k_ % 2])

        def wait_store(k_):
            pltpu.make_async_copy(
                accs[k_ % 2], out_hbm.at[pl.ds(obase(k_), SC_CHUNK)],
                sems[k_ % 2]).wait()

        for k_ in range(per_worker):
            @pl.when(valid(k_))
            def _(k_=k_):
                pltpu.async_copy(
                    idx_hbm.at[cid(k_)],
                    idx_all.at[pl.ds(k_ * NIDX, NIDX)], semx)
        for k_ in range(per_worker):
            @pl.when(valid(k_))
            def _(k_=k_):
                pltpu.make_async_copy(
                    idx_hbm.at[cid(k_)],
                    idx_all.at[pl.ds(k_ * NIDX, NIDX)], semx).wait()

        @pl.when(valid(0))
        def _():
            init_gather(0)

        for k_ in range(per_worker):
            p = k_ % 2

            @pl.when(valid(k_))
            def _(k_=k_, p=p):
                wait_init(k_)
                for j in range(1, MAX_NB):
                    pltpu.async_copy(
                        msg_hbm.at[idx_slice(k_, j)], accs[p], sema[p],
                        add=True)

            if k_ + 1 < per_worker:
                @pl.when(valid(k_ + 1))
                def _(k_=k_):
                    if k_ >= 1:
                        wait_store(k_ - 1)
                    init_gather(k_ + 1)

            @pl.when(valid(k_))
            def _(k_=k_, p=p):
                for j in range(1, MAX_NB):
                    pltpu.make_async_copy(
                        msg_hbm.at[idx_slice(k_, j)], accs[p],
                        sema[p]).wait()
                store(k_)

        for k_ in range(per_worker):
            @pl.when(valid(k_) & (cid(k_) + 2 * NUM_WORKERS >= chunk_hi))
            def _(k_=k_):
                wait_store(k_)

    return k(message, idx_c)


def _mm_init(fbonds_p, W_i):
    grid = N_BONDS_PAD // BOND_BLK

    def body(f_ref, w_ref, bin_ref, msg_ref):
        acc = lax.dot_general(f_ref[...], w_ref[...],
                              (((1,), (1,)), ((), ())),
                              preferred_element_type=jnp.float32)
        bin_ref[...] = acc.astype(jnp.bfloat16)
        msg_ref[...] = jnp.maximum(acc, 0.0)

    return pl.pallas_call(
        body,
        grid=(grid,),
        in_specs=[
            pl.BlockSpec((BOND_BLK, ATOM_FDIM + BOND_FDIM),
                         lambda i: (i, 0)),
            pl.BlockSpec((HIDDEN, ATOM_FDIM + BOND_FDIM), lambda i: (0, 0)),
        ],
        out_specs=[
            pl.BlockSpec((BOND_BLK, HIDDEN), lambda i: (i, 0)),
            pl.BlockSpec((BOND_BLK, HIDDEN), lambda i: (i, 0)),
        ],
        out_shape=[
            jax.ShapeDtypeStruct((N_BONDS_PAD, HIDDEN), jnp.bfloat16),
            jax.ShapeDtypeStruct((N_BONDS_PAD, HIDDEN), jnp.float32),
        ],
    )(fbonds_p, W_i)


def _mm_h_part(nei_part, binput, W_h, blk_off, prev=None):
    grid = nei_part.shape[0] // BOND_BLK

    def body(*refs):
        n_ref, b_ref, w_ref = refs[0], refs[1], refs[2]
        o_ref = refs[-1]
        acc = lax.dot_general(n_ref[...], w_ref[...],
                              (((1,), (1,)), ((), ())),
                              preferred_element_type=jnp.float32)
        o_ref[...] = jnp.maximum(b_ref[...].astype(jnp.float32) + acc, 0.0)

    in_specs = [
        pl.BlockSpec((BOND_BLK, HIDDEN), lambda i: (i, 0)),
        pl.BlockSpec((BOND_BLK, HIDDEN), lambda i: (i + blk_off, 0)),
        pl.BlockSpec((HIDDEN, HIDDEN), lambda i: (0, 0)),
    ]
    args = [nei_part, binput, W_h]
    aliases = {}
    if prev is not None:
        in_specs.append(pl.BlockSpec((8, HIDDEN), lambda i: (0, 0)))
        args.append(prev)
        aliases = {3: 0}

    return pl.pallas_call(
        body,
        grid=(grid,),
        in_specs=in_specs,
        out_specs=pl.BlockSpec((BOND_BLK, HIDDEN),
                               lambda i: (i + blk_off, 0)),
        out_shape=jax.ShapeDtypeStruct((N_BONDS_PAD, HIDDEN), jnp.float32),
        input_output_aliases=aliases,
    )(*args)


def _final_part(fatoms_p, nei_part, W_oa, W_om, W_o_b, blk_off, prev=None):
    grid = nei_part.shape[0] // ATOM_BLK
    mb = ATOM_BLK // MOL_LEN

    def body(*refs):
        f_ref, n_ref, wa_ref, wm_ref, b_ref = refs[:5]
        o_ref = refs[-1]
        h = lax.dot_general(f_ref[...], wa_ref[...],
                            (((1,), (1,)), ((), ())),
                            preferred_element_type=jnp.float32)
        h = h + lax.dot_general(n_ref[...], wm_ref[...],
                                (((1,), (1,)), ((), ())),
                                preferred_element_type=jnp.float32)
        h = jnp.maximum(h + b_ref[...], 0.0)
        mrow = lax.broadcasted_iota(jnp.int32, (mb, ATOM_BLK), 0)
        acol = lax.broadcasted_iota(jnp.int32, (mb, ATOM_BLK), 1) // MOL_LEN
        P = jnp.where(mrow == acol, 1.0 / MOL_LEN, 0.0).astype(jnp.float32)
        o_ref[...] = lax.dot_general(P, h, (((1,), (0,)), ((), ())),
                                     preferred_element_type=jnp.float32)

    in_specs = [
        pl.BlockSpec((ATOM_BLK, ATOM_FDIM), lambda i: (i + blk_off, 0)),
        pl.BlockSpec((ATOM_BLK, HIDDEN), lambda i: (i, 0)),
        pl.BlockSpec((HIDDEN, ATOM_FDIM), lambda i: (0, 0)),
        pl.BlockSpec((HIDDEN, HIDDEN), lambda i: (0, 0)),
        pl.BlockSpec((1, HIDDEN), lambda i: (0, 0)),
    ]
    args = [fatoms_p, nei_part, W_oa, W_om, W_o_b]
    aliases = {}
    if prev is not None:
        in_specs.append(pl.BlockSpec((8, HIDDEN), lambda i: (0, 0)))
        args.append(prev)
        aliases = {5: 0}

    return pl.pallas_call(
        body,
        grid=(grid,),
        in_specs=in_specs,
        out_specs=pl.BlockSpec((mb, HIDDEN), lambda i: (i + blk_off, 0)),
        out_shape=jax.ShapeDtypeStruct((N_MOLS_PAD, HIDDEN), jnp.float32),
        input_output_aliases=aliases,
    )(*args)


def _pad_graph(graph, n_rows_pad, n_table):
    pad = n_rows_pad - graph.shape[0]
    filler = jnp.arange(pad * MAX_NB, dtype=jnp.int32) % n_table
    return jnp.concatenate([graph, filler.reshape(pad, MAX_NB)], axis=0)


def _chunk_indices(g):
    g = g.reshape(g.shape[0] // SC_CHUNK, SC_CHUNK, MAX_NB)
    return g.transpose(0, 2, 1).reshape(-1, NIDX)


def kernel(fatoms, fbonds, agraph, bgraph, scope, W_i, W_h, W_o_w, W_o_b):
    del scope
    g_all = jnp.concatenate([
        _pad_graph(bgraph, N_BONDS_PAD, N_BONDS),
        _pad_graph(agraph, N_ATOMS_PAD, N_BONDS)], axis=0)
    gidx = _chunk_indices(g_all)
    fbonds_p = jnp.pad(fbonds, ((0, N_BONDS_PAD - N_BONDS), (0, 0)))
    fatoms_p = jnp.pad(fatoms, ((0, N_ATOMS_PAD - N_ATOMS), (0, 0)))

    n_bchunks = N_BONDS_PAD // SC_CHUNK
    b_blk_off = (B_SPLIT * SC_CHUNK) // BOND_BLK
    a_blk_off = (A_SPLIT * SC_CHUNK) // ATOM_BLK

    bidx0, bidx1 = gidx[:B_SPLIT], gidx[B_SPLIT:n_bchunks]
    aidx0 = gidx[n_bchunks:n_bchunks + A_SPLIT]
    aidx1 = gidx[n_bchunks + A_SPLIT:]
    binput, message = _mm_init(fbonds_p, W_i)
    for _ in range(DEPTH - 1):
        n0 = _gather_sum_sc(message, bidx0)
        n1 = _gather_sum_sc(message, bidx1)
        m0 = _mm_h_part(n0, binput, W_h, 0)
        message = _mm_h_part(n1, binput, W_h, b_blk_off, prev=m0)
    a0 = _gather_sum_sc(message, aidx0)
    a1 = _gather_sum_sc(message, aidx1)
    W_oa = W_o_w[:, :ATOM_FDIM]
    W_om = W_o_w[:, ATOM_FDIM:]
    bias = W_o_b.reshape(1, HIDDEN)
    f0 = _final_part(fatoms_p, a0, W_oa, W_om, bias, 0)
    f1 = _final_part(fatoms_p, a1, W_oa, W_om, bias, a_blk_off, prev=f0)
    return f1[:N_MOLS]
